# Initial kernel scaffold; baseline (speedup 1.0000x reference)
import jax
import jax.numpy as jnp
from jax.experimental import pallas as pl


def kernel(x, embeddings):
    raise NotImplementedError("write your pallas kernel here")


# speedup vs baseline: 5.0486x; 5.0486x over previous
import functools

import jax
import jax.numpy as jnp
from jax import lax
from jax.experimental import pallas as pl
from jax.experimental.pallas import tpu as pltpu
from jax.experimental.pallas import tpu_sc as plsc

N = 16384
L = 200
D = 32
B = N * L

_info = plsc.get_sparse_core_info()
NC = _info.num_cores
NS = _info.num_subcores
NW = NC * NS
B_PER_W = B // NW
CHUNK = 1024
STEPS = B_PER_W // CHUNK
NBUF = 2
MAIN_GROUPS = (STEPS - NBUF) // NBUF


@functools.partial(
    pl.kernel,
    mesh=plsc.VectorSubcoreMesh(core_axis_name="c", subcore_axis_name="s"),
    out_type=jax.ShapeDtypeStruct((B, D), jnp.float32),
    scratch_types=[
        pltpu.VMEM((CHUNK,), jnp.int32),
        pltpu.VMEM((CHUNK,), jnp.int32),
        pltpu.VMEM((CHUNK, D), jnp.float32),
        pltpu.VMEM((CHUNK, D), jnp.float32),
        pltpu.SemaphoreType.DMA,
        pltpu.SemaphoreType.DMA,
        pltpu.SemaphoreType.DMA,
    ],
    compiler_params=pltpu.CompilerParams(use_tc_tiling_on_sc=False),
)
def _gather_kernel(idx_hbm, table_hbm, out_hbm, idx0, idx1, rows0, rows1,
                   gsem, osem, isem):
    idxs = (idx0, idx1)
    rows = (rows0, rows1)
    wid = lax.axis_index("s") * NC + lax.axis_index("c")
    base = wid * B_PER_W

    def off(c):
        return base + c * CHUNK

    for b in range(NBUF):
        pltpu.sync_copy(idx_hbm.at[pl.ds(off(b), CHUNK)], idxs[b])
        pltpu.async_copy(table_hbm.at[idxs[b]], rows[b], gsem)

    def group(g, carry):
        for b in range(NBUF):
            c = g * NBUF + b
            pltpu.make_async_copy(table_hbm.at[idxs[b]], rows[b], gsem).wait()
            pltpu.async_copy(rows[b], out_hbm.at[pl.ds(off(c), CHUNK)], osem)
            pltpu.async_copy(
                idx_hbm.at[pl.ds(off(c + NBUF), CHUNK)], idxs[b], isem)
            pltpu.make_async_copy(
                rows[b], out_hbm.at[pl.ds(off(c), CHUNK)], osem).wait()
            pltpu.make_async_copy(
                idx_hbm.at[pl.ds(off(c + NBUF), CHUNK)], idxs[b], isem).wait()
            pltpu.async_copy(table_hbm.at[idxs[b]], rows[b], gsem)
        return carry

    lax.fori_loop(0, MAIN_GROUPS, group, 0)

    for b in range(NBUF):
        c = STEPS - NBUF + b
        pltpu.make_async_copy(table_hbm.at[idxs[b]], rows[b], gsem).wait()
        pltpu.async_copy(rows[b], out_hbm.at[pl.ds(off(c), CHUNK)], osem)
    for b in range(NBUF):
        c = STEPS - NBUF + b
        pltpu.make_async_copy(
            rows[b], out_hbm.at[pl.ds(off(c), CHUNK)], osem).wait()


def kernel(x, embeddings):
    flat = x.reshape(B).astype(jnp.int32)
    out = _gather_kernel(flat, embeddings)
    return out.reshape(N, L, D)
